# baseline (speedup 1.0000x reference)
import functools

import jax
import jax.numpy as jnp
from jax.experimental import pallas as pl

_R_CUT = 5.0
_L = 4
_A = 4
_NSP = 8
_NTOT = 128


def _sactivate(y):
    return y * jnp.tanh(y) + y


def _sinpi(t):
    t2 = t * t
    p = jnp.float32(-0.5517513410677957)
    p = p * t2 + jnp.float32(2.5406914267260223)
    p = p * t2 + jnp.float32(-5.166999911630681)
    p = p * t2 + jnp.float32(3.1415778644187387)
    return p * t


def _fwd(r_ref, s_ref, wc_ref, w1_ref, w2_ref, w3_ref, w4_ref, out_ref):
    eb = r_ref.shape[0]
    x = r_ref[:, :]
    u = jnp.clip(x, 0.0, _R_CUT) * jnp.float32(1.0 / _R_CUT)
    cutoff_half = 0.25 * (_sinpi(0.5 - u) + 1.0)

    ki = jax.lax.broadcasted_iota(jnp.int32, (eb, _NTOT), 1) + 1
    k = ki.astype(jnp.float32)
    ku = k * u
    n = jnp.floor(ku + 0.5)
    f = ku - n
    half = n * 0.5
    sgn = 1.0 - 4.0 * (half - jnp.floor(half))
    rf = _sinpi(f) * sgn

    s = s_ref[:, :]
    sp = jax.lax.broadcasted_iota(jnp.int32, (eb, _NSP), 1)
    onehot = (s == sp).astype(jnp.float32)
    co = onehot * cutoff_half

    for a in range(_A):
        g = jnp.dot(co, wc_ref[a], preferred_element_type=jnp.float32,
                    precision=jax.lax.Precision.HIGHEST)
        y = jnp.dot(rf, w1_ref[a], preferred_element_type=jnp.float32) * g
        h = _sactivate(y)
        h = _sactivate(jnp.dot(h, w2_ref[a], preferred_element_type=jnp.float32))
        h = _sactivate(jnp.dot(h, w3_ref[a], preferred_element_type=jnp.float32))
        o = jnp.dot(h, w4_ref[a], preferred_element_type=jnp.float32)
        out_ref[:, a, :] = o


def _block_diag_t(w, scale):
    wt = jnp.transpose(w, (1, 0, 3, 2)) * scale
    eye = jnp.eye(_L, dtype=jnp.float32)
    m = wt[:, :, :, None, :] * eye[None, :, None, :, None]
    return m.reshape(_A, _NTOT, _NTOT)


@functools.partial(jax.jit, static_argnames=())
def kernel(r, species_neighbor, w_comb, mlp_w1, mlp_w2, mlp_w3, mlp_w4):
    e = r.shape[0]
    eb = 2000
    grid = pl.cdiv(e, eb)
    w1 = _block_diag_t(mlp_w1, 1.0)
    w2 = _block_diag_t(mlp_w2, 0.5)
    w3 = _block_diag_t(mlp_w3, 0.5)
    w4 = _block_diag_t(mlp_w4, 1.0)
    wbc = jnp.broadcast_to(w_comb[:, :, None], (_A, _NSP, _NTOT))
    r2 = r.astype(jnp.float32).reshape(e, 1)
    s2 = species_neighbor.astype(jnp.int32).reshape(e, 1)
    full = lambda i: (0, 0, 0)
    out = pl.pallas_call(
        _fwd,
        grid=(grid,),
        in_specs=[
            pl.BlockSpec((eb, 1), lambda i: (i, 0)),
            pl.BlockSpec((eb, 1), lambda i: (i, 0)),
            pl.BlockSpec((_A, _NSP, _NTOT), full),
            pl.BlockSpec((_A, _NTOT, _NTOT), full),
            pl.BlockSpec((_A, _NTOT, _NTOT), full),
            pl.BlockSpec((_A, _NTOT, _NTOT), full),
            pl.BlockSpec((_A, _NTOT, _NTOT), full),
        ],
        out_specs=pl.BlockSpec((eb, _A, _NTOT), lambda i: (i, 0, 0)),
        out_shape=jax.ShapeDtypeStruct((e, _A, _NTOT), jnp.float32),
    )(r2, s2, wbc, w1, w2, w3, w4)
    return out


# speedup vs baseline: 1.7123x; 1.7123x over previous
import functools

import jax
import jax.numpy as jnp
from jax.experimental import pallas as pl
from jax.experimental.pallas import tpu as pltpu

_R_CUT = 5.0
_L = 4
_A = 4
_NSP = 8
_NTOT = 128


def _sactivate(y):
    return y * jnp.tanh(y) + y


def _sinpi(t):
    t2 = t * t
    p = jnp.float32(-0.5517513410677957)
    p = p * t2 + jnp.float32(2.5406914267260223)
    p = p * t2 + jnp.float32(-5.166999911630681)
    p = p * t2 + jnp.float32(3.1415778644187387)
    return p * t


def _fwd(r_ref, s_ref, wc_ref, w1_ref, w2_ref, w3_ref, w4_ref, out_ref):
    eb = r_ref.shape[0]
    x = r_ref[:, :]
    u = jnp.clip(x, 0.0, _R_CUT) * jnp.float32(1.0 / _R_CUT)
    cutoff_half = 0.25 * (_sinpi(0.5 - u) + 1.0)

    ki = jax.lax.broadcasted_iota(jnp.int32, (eb, _NTOT), 1) + 1
    k = ki.astype(jnp.float32)
    ku = k * u
    n = jnp.floor(ku + 0.5)
    f = ku - n
    half = n * 0.5
    sgn = 1.0 - 4.0 * (half - jnp.floor(half))
    rf = _sinpi(f) * sgn

    s = s_ref[:, :]
    sp = jax.lax.broadcasted_iota(jnp.int32, (eb, _NSP), 1)
    onehot = (s == sp).astype(jnp.float32)
    psw = jnp.dot(onehot, wc_ref[:, :].T,
                  preferred_element_type=jnp.float32)
    gamma = psw * cutoff_half

    for a in range(_A):
        g = gamma[:, a][:, None]
        y = jnp.dot(rf, w1_ref[a], preferred_element_type=jnp.float32) * g
        h = _sactivate(y)
        h = _sactivate(jnp.dot(h, w2_ref[a], preferred_element_type=jnp.float32))
        h = _sactivate(jnp.dot(h, w3_ref[a], preferred_element_type=jnp.float32))
        o = jnp.dot(h, w4_ref[a], preferred_element_type=jnp.float32)
        out_ref[:, a, :] = o


def _block_diag_t(w, scale):
    wt = jnp.transpose(w, (1, 0, 3, 2)) * scale
    eye = jnp.eye(_L, dtype=jnp.float32)
    m = wt[:, :, :, None, :] * eye[None, :, None, :, None]
    return m.reshape(_A, _NTOT, _NTOT)


@functools.partial(jax.jit, static_argnames=())
def kernel(r, species_neighbor, w_comb, mlp_w1, mlp_w2, mlp_w3, mlp_w4):
    e = r.shape[0]
    eb = 2000
    grid = pl.cdiv(e, eb)
    w1 = _block_diag_t(mlp_w1, 1.0)
    w2 = _block_diag_t(mlp_w2, 0.5)
    w3 = _block_diag_t(mlp_w3, 0.5)
    w4 = _block_diag_t(mlp_w4, 1.0)
    r2 = r.astype(jnp.float32).reshape(e, 1)
    s2 = species_neighbor.astype(jnp.int32).reshape(e, 1)
    full = lambda i: (0, 0, 0)
    out = pl.pallas_call(
        _fwd,
        grid=(grid,),
        in_specs=[
            pl.BlockSpec((eb, 1), lambda i: (i, 0)),
            pl.BlockSpec((eb, 1), lambda i: (i, 0)),
            pl.BlockSpec((_A, _NSP), lambda i: (0, 0)),
            pl.BlockSpec((_A, _NTOT, _NTOT), full),
            pl.BlockSpec((_A, _NTOT, _NTOT), full),
            pl.BlockSpec((_A, _NTOT, _NTOT), full),
            pl.BlockSpec((_A, _NTOT, _NTOT), full),
        ],
        out_specs=pl.BlockSpec((eb, _A, _NTOT), lambda i: (i, 0, 0)),
        out_shape=jax.ShapeDtypeStruct((e, _A, _NTOT), jnp.float32),
        compiler_params=pltpu.CompilerParams(
            dimension_semantics=("parallel",)),
    )(r2, s2, w_comb, w1, w2, w3, w4)
    return out


# speedup vs baseline: 1.7187x; 1.0037x over previous
import functools

import jax
import jax.numpy as jnp
from jax.experimental import pallas as pl

_R_CUT = 5.0
_L = 4
_A = 4
_NSP = 8
_NTOT = 128


def _sactivate(y):
    return y * jnp.tanh(y) + y


def _sinpi(t):
    t2 = t * t
    p = jnp.float32(-0.5517513410677957)
    p = p * t2 + jnp.float32(2.5406914267260223)
    p = p * t2 + jnp.float32(-5.166999911630681)
    p = p * t2 + jnp.float32(3.1415778644187387)
    return p * t


def _fwd(r_ref, s_ref, wc_ref, w1_ref, w2_ref, w3_ref, w4_ref, out_ref):
    eb = r_ref.shape[0]
    x = r_ref[:, :]
    u = jnp.clip(x, 0.0, _R_CUT) * jnp.float32(1.0 / _R_CUT)
    cutoff_half = 0.25 * (_sinpi(0.5 - u) + 1.0)

    ki = jax.lax.broadcasted_iota(jnp.int32, (eb, _NTOT), 1) + 1
    k = ki.astype(jnp.float32)
    ku = k * u
    n = jnp.floor(ku + 0.5)
    f = ku - n
    half = n * 0.5
    sgn = 1.0 - 4.0 * (half - jnp.floor(half))
    rf = _sinpi(f) * sgn

    s = s_ref[:, :]
    sp = jax.lax.broadcasted_iota(jnp.int32, (eb, _NSP), 1)
    onehot = (s == sp).astype(jnp.float32)
    psw = jnp.dot(onehot, wc_ref[:, :].T,
                  preferred_element_type=jnp.float32)
    gamma = psw * cutoff_half

    for a in range(_A):
        g = gamma[:, a][:, None]
        y = jnp.dot(rf, w1_ref[a], preferred_element_type=jnp.float32) * g
        h = _sactivate(y)
        h = _sactivate(jnp.dot(h, w2_ref[a], preferred_element_type=jnp.float32))
        h = _sactivate(jnp.dot(h, w3_ref[a], preferred_element_type=jnp.float32))
        o = jnp.dot(h, w4_ref[a], preferred_element_type=jnp.float32)
        out_ref[:, a, :] = o


def _block_diag_t(w, scale):
    wt = jnp.transpose(w, (1, 0, 3, 2)) * scale
    eye = jnp.eye(_L, dtype=jnp.float32)
    m = wt[:, :, :, None, :] * eye[None, :, None, :, None]
    return m.reshape(_A, _NTOT, _NTOT)


@functools.partial(jax.jit, static_argnames=())
def kernel(r, species_neighbor, w_comb, mlp_w1, mlp_w2, mlp_w3, mlp_w4):
    e = r.shape[0]
    eb = 2000
    grid = pl.cdiv(e, eb)
    w1 = _block_diag_t(mlp_w1, 1.0)
    w2 = _block_diag_t(mlp_w2, 0.5)
    w3 = _block_diag_t(mlp_w3, 0.5)
    w4 = _block_diag_t(mlp_w4, 1.0)
    r2 = r.astype(jnp.float32).reshape(e, 1)
    s2 = species_neighbor.astype(jnp.int32).reshape(e, 1)
    full = lambda i: (0, 0, 0)
    out = pl.pallas_call(
        _fwd,
        grid=(grid,),
        in_specs=[
            pl.BlockSpec((eb, 1), lambda i: (i, 0)),
            pl.BlockSpec((eb, 1), lambda i: (i, 0)),
            pl.BlockSpec((_A, _NSP), lambda i: (0, 0)),
            pl.BlockSpec((_A, _NTOT, _NTOT), full),
            pl.BlockSpec((_A, _NTOT, _NTOT), full),
            pl.BlockSpec((_A, _NTOT, _NTOT), full),
            pl.BlockSpec((_A, _NTOT, _NTOT), full),
        ],
        out_specs=pl.BlockSpec((eb, _A, _NTOT), lambda i: (i, 0, 0)),
        out_shape=jax.ShapeDtypeStruct((e, _A, _NTOT), jnp.float32),
    )(r2, s2, w_comb, w1, w2, w3, w4)
    return out
